# baseline (speedup 1.0000x reference)
import functools

import jax
import jax.numpy as jnp
from jax.experimental import pallas as pl
from jax.experimental.pallas import tpu as pltpu

MAX_DEG = 6
N = 100000
F = 128
BATCH = 2000
DEG_SIZES = [4000, 16000, 16000, 16000, 16000, 16000, 16000]
DEG_STARTS = [0, 4000, 20000, 36000, 52000, 68000, 84000]


def _graph_conv(atoms, deg_adj_lists, W, b):
    rel = [None] * (MAX_DEG + 1)
    widx = 0
    for deg in range(1, MAX_DEG + 1):
        gathered = jnp.take(atoms, deg_adj_lists[deg - 1], axis=0)
        summed = jnp.sum(gathered, axis=1)
        out = summed @ W[widx] + b[widx]
        widx += 1
        self_atoms = atoms[DEG_STARTS[deg]:DEG_STARTS[deg] + DEG_SIZES[deg]]
        out = out + self_atoms @ W[widx] + b[widx]
        widx += 1
        rel[deg] = out
    self_atoms = atoms[DEG_STARTS[0]:DEG_STARTS[0] + DEG_SIZES[0]]
    rel[0] = self_atoms @ W[widx] + b[widx]
    return jnp.tanh(jnp.concatenate(rel, axis=0))


def _graph_pool(atoms, deg_adj_lists):
    rel = [None] * (MAX_DEG + 1)
    for deg in range(1, MAX_DEG + 1):
        self_atoms = atoms[DEG_STARTS[deg]:DEG_STARTS[deg] + DEG_SIZES[deg]]
        gathered = jnp.take(atoms, deg_adj_lists[deg - 1], axis=0)
        maxed = jnp.max(jnp.concatenate([self_atoms[:, None, :], gathered], axis=1), axis=1)
        rel[deg] = maxed
    rel[0] = atoms[DEG_STARTS[0]:DEG_STARTS[0] + DEG_SIZES[0]]
    return jnp.concatenate(rel, axis=0)


def _batch_norm(x, gamma, beta, eps=1e-3):
    m = jnp.mean(x, axis=0, keepdims=True)
    v = jnp.var(x, axis=0, keepdims=True)
    return gamma * (x - m) / jnp.sqrt(v + eps) + beta


def _final_body(s_ref, m_ref, d2w_ref, d2b_ref, d3w_ref, d3b_ref, out_ref):
    g = jnp.tanh(jnp.concatenate([s_ref[...], m_ref[...]], axis=1))
    z = jax.nn.sigmoid(g @ d2w_ref[...] + d2b_ref[...])
    out_ref[...] = z @ d3w_ref[...] + d3b_ref[...]


def kernel(atom_features, degree_slice, membership, deg_adj_1, deg_adj_2, deg_adj_3, deg_adj_4, deg_adj_5, deg_adj_6, gc1_W, gc1_b, gc2_W, gc2_b, bn1_gamma, bn1_beta, bn3_gamma, bn3_beta, d1_W, d1_b, d2_W, d2_b, d3_W, d3_b):
    deg_adj_lists = [deg_adj_1, deg_adj_2, deg_adj_3, deg_adj_4, deg_adj_5, deg_adj_6]
    h = _graph_conv(atom_features, deg_adj_lists, gc1_W, gc1_b)
    h = _batch_norm(h, bn1_gamma, bn1_beta)
    h = _graph_pool(h, deg_adj_lists)
    h = _graph_conv(h, deg_adj_lists, gc2_W, gc2_b)
    h = _batch_norm(h, bn1_gamma, bn1_beta)
    h = _graph_pool(h, deg_adj_lists)
    h = jnp.tanh(h @ d1_W + d1_b)
    h = _batch_norm(h, bn3_gamma, bn3_beta)
    s = jax.ops.segment_sum(h, membership, num_segments=BATCH)
    m = jax.ops.segment_max(h, membership, num_segments=BATCH)
    out = pl.pallas_call(
        _final_body,
        out_shape=jax.ShapeDtypeStruct((BATCH, 1), jnp.float32),
    )(s, m, d2_W, d2_b, d3_W, d3_b)
    return out


# speedup vs baseline: 1.1804x; 1.1804x over previous
import functools

import jax
import jax.numpy as jnp
from jax import lax
from jax.experimental import pallas as pl
from jax.experimental.pallas import tpu as pltpu
from jax.experimental.pallas import tpu_sc as plsc

MAX_DEG = 6
N = 100000
F = 128
BATCH = 2000
DEG_SIZES = [4000, 16000, 16000, 16000, 16000, 16000, 16000]
DEG_STARTS = [0, 4000, 20000, 36000, 52000, 68000, 84000]
NW = 32
ROWS_PER_TILE = 500

C_D = {1: 128, 2: 128, 3: 120, 4: 128, 5: 120, 6: 120}
R_D = {d: C_D[d] // d for d in C_D}
NCHUNK = {d: -(-ROWS_PER_TILE // R_D[d]) for d in C_D}
P_D = {d: NCHUNK[d] * C_D[d] for d in C_D}
SEG_OFF = {}
_off = 0
for _d in range(1, 7):
    SEG_OFF[_d] = _off
    _off += NW * P_D[_d]
IDX_TOTAL = _off


def _build_padded_idx(adj_lists):
    segs = []
    for d in range(1, 7):
        a = adj_lists[d - 1].reshape(NW, ROWS_PER_TILE * d)
        a = jnp.pad(a, ((0, 0), (0, P_D[d] - ROWS_PER_TILE * d)))
        segs.append(a.reshape(-1))
    return jnp.concatenate(segs)


def _make_gather_sum(Fc):
    mesh = plsc.VectorSubcoreMesh(core_axis_name="c", subcore_axis_name="s")

    @functools.partial(
        pl.kernel,
        out_type=jax.ShapeDtypeStruct((N, Fc), jnp.float32),
        mesh=mesh,
        compiler_params=pltpu.CompilerParams(use_tc_tiling_on_sc=False),
        scratch_types=[
            pltpu.VMEM((128,), jnp.int32),
            pltpu.VMEM((120,), jnp.int32),
            pltpu.VMEM((128, Fc), jnp.float32),
            pltpu.VMEM((128, Fc), jnp.float32),
            pltpu.SemaphoreType.DMA,
        ],
    )
    def gather_sum(table, idxp, ns, idx128, idx120, g_v, o_v, sem):
        wid = lax.axis_index("s") * 2 + lax.axis_index("c")
        zero = jnp.zeros((16,), jnp.float32)

        def zbody(r, carry):
            for f in range(Fc // 16):
                o_v[r, pl.ds(16 * f, 16)] = zero
            return carry

        lax.fori_loop(0, 125, zbody, 0)
        pltpu.sync_copy(o_v.at[pl.ds(0, 125)], ns.at[pl.ds(wid * 125, 125)])

        for d in range(1, 7):
            C, R = C_D[d], R_D[d]
            idx_v = idx128 if C == 128 else idx120
            base_idx = SEG_OFF[d] + wid * P_D[d]
            row0 = DEG_STARTS[d] + wid * ROWS_PER_TILE
            nfull = ROWS_PER_TILE // R
            rem = ROWS_PER_TILE - nfull * R

            def do_chunk(c, rows_c, d=d, C=C, R=R, base_idx=base_idx, row0=row0,
                         idx_v=idx_v):
                pltpu.sync_copy(idxp.at[pl.ds(base_idx + c * C, C)], idx_v)
                pltpu.async_copy(table.at[idx_v], g_v.at[pl.ds(0, C)], sem).wait()

                def body(r, carry):
                    for f in range(Fc // 16):
                        sl = pl.ds(16 * f, 16)
                        acc = g_v[r * d, sl]
                        for j in range(1, d):
                            acc = acc + g_v[r * d + j, sl]
                        o_v[r, sl] = acc
                    return carry

                lax.fori_loop(0, rows_c, body, 0)
                pltpu.sync_copy(o_v.at[pl.ds(0, rows_c)],
                                ns.at[pl.ds(row0 + c * R, rows_c)])

            def chunk_body(c, carry, do_chunk=do_chunk, R=R):
                do_chunk(c, R)
                return carry

            lax.fori_loop(0, nfull, chunk_body, 0)
            if rem:
                do_chunk(nfull, rem)

    return gather_sum


_gather_sum_128 = _make_gather_sum(128)
_gather_sum_64 = _make_gather_sum(64)


def _graph_pool(atoms, deg_adj_lists):
    rel = [None] * (MAX_DEG + 1)
    for deg in range(1, MAX_DEG + 1):
        self_atoms = atoms[DEG_STARTS[deg]:DEG_STARTS[deg] + DEG_SIZES[deg]]
        gathered = jnp.take(atoms, deg_adj_lists[deg - 1], axis=0)
        maxed = jnp.max(jnp.concatenate([self_atoms[:, None, :], gathered], axis=1), axis=1)
        rel[deg] = maxed
    rel[0] = atoms[DEG_STARTS[0]:DEG_STARTS[0] + DEG_SIZES[0]]
    return jnp.concatenate(rel, axis=0)


def _batch_norm(x, gamma, beta, eps=1e-3):
    m = jnp.mean(x, axis=0, keepdims=True)
    v = jnp.var(x, axis=0, keepdims=True)
    return gamma * (x - m) / jnp.sqrt(v + eps) + beta


def _graph_conv(atoms, ns, W, b):
    rel = [None] * (MAX_DEG + 1)
    widx = 0
    for deg in range(1, MAX_DEG + 1):
        summed = ns[DEG_STARTS[deg]:DEG_STARTS[deg] + DEG_SIZES[deg]]
        out = summed @ W[widx] + b[widx]
        widx += 1
        self_atoms = atoms[DEG_STARTS[deg]:DEG_STARTS[deg] + DEG_SIZES[deg]]
        out = out + self_atoms @ W[widx] + b[widx]
        widx += 1
        rel[deg] = out
    self_atoms = atoms[DEG_STARTS[0]:DEG_STARTS[0] + DEG_SIZES[0]]
    rel[0] = self_atoms @ W[widx] + b[widx]
    return jnp.tanh(jnp.concatenate(rel, axis=0))


def _final_body(s_ref, m_ref, d2w_ref, d2b_ref, d3w_ref, d3b_ref, out_ref):
    g = jnp.tanh(jnp.concatenate([s_ref[...], m_ref[...]], axis=1))
    z = jax.nn.sigmoid(g @ d2w_ref[...] + d2b_ref[...])
    out_ref[...] = z @ d3w_ref[...] + d3b_ref[...]


def kernel(atom_features, degree_slice, membership, deg_adj_1, deg_adj_2, deg_adj_3, deg_adj_4, deg_adj_5, deg_adj_6, gc1_W, gc1_b, gc2_W, gc2_b, bn1_gamma, bn1_beta, bn3_gamma, bn3_beta, d1_W, d1_b, d2_W, d2_b, d3_W, d3_b):
    deg_adj_lists = [deg_adj_1, deg_adj_2, deg_adj_3, deg_adj_4, deg_adj_5, deg_adj_6]
    idxp = _build_padded_idx(deg_adj_lists)

    ns1 = _gather_sum_128(atom_features, idxp)
    h = _graph_conv(atom_features, ns1, gc1_W, gc1_b)
    h = _batch_norm(h, bn1_gamma, bn1_beta)
    h = _graph_pool(h, deg_adj_lists)
    ns2 = _gather_sum_64(h, idxp)
    h = _graph_conv(h, ns2, gc2_W, gc2_b)
    h = _batch_norm(h, bn1_gamma, bn1_beta)
    h = _graph_pool(h, deg_adj_lists)
    h = jnp.tanh(h @ d1_W + d1_b)
    h = _batch_norm(h, bn3_gamma, bn3_beta)
    s = jax.ops.segment_sum(h, membership, num_segments=BATCH)
    m = jax.ops.segment_max(h, membership, num_segments=BATCH)
    out = pl.pallas_call(
        _final_body,
        out_shape=jax.ShapeDtypeStruct((BATCH, 1), jnp.float32),
    )(s, m, d2_W, d2_b, d3_W, d3_b)
    return out


# speedup vs baseline: 1.9713x; 1.6700x over previous
import functools

import jax
import jax.numpy as jnp
from jax import lax
from jax.experimental import pallas as pl
from jax.experimental.pallas import tpu as pltpu
from jax.experimental.pallas import tpu_sc as plsc

MAX_DEG = 6
N = 100000
F = 128
BATCH = 2000
DEG_SIZES = [4000, 16000, 16000, 16000, 16000, 16000, 16000]
DEG_STARTS = [0, 4000, 20000, 36000, 52000, 68000, 84000]
NW = 32
ROWS_PER_TILE = 500

C_D = {1: 128, 2: 128, 3: 120, 4: 128, 5: 120, 6: 120}
R_D = {d: C_D[d] // d for d in C_D}
NCHUNK = {d: -(-ROWS_PER_TILE // R_D[d]) for d in C_D}
P_D = {d: NCHUNK[d] * C_D[d] for d in C_D}
SEG_OFF = {}
_off = 0
for _d in range(1, 7):
    SEG_OFF[_d] = _off
    _off += NW * P_D[_d]
IDX_TOTAL = _off


def _build_padded_idx(adj_lists):
    segs = []
    for d in range(1, 7):
        a = adj_lists[d - 1].reshape(NW, ROWS_PER_TILE * d)
        a = jnp.pad(a, ((0, 0), (0, P_D[d] - ROWS_PER_TILE * d)))
        segs.append(a.reshape(-1))
    return jnp.concatenate(segs)


def _make_gather_sum(Fc):
    mesh = plsc.VectorSubcoreMesh(core_axis_name="c", subcore_axis_name="s")

    @functools.partial(
        pl.kernel,
        out_type=jax.ShapeDtypeStruct((N, Fc), jnp.float32),
        mesh=mesh,
        compiler_params=pltpu.CompilerParams(use_tc_tiling_on_sc=False),
        scratch_types=[
            pltpu.VMEM((128,), jnp.int32),
            pltpu.VMEM((120,), jnp.int32),
            pltpu.VMEM((128, Fc), jnp.float32),
            pltpu.VMEM((128, Fc), jnp.float32),
            pltpu.SemaphoreType.DMA,
        ],
    )
    def gather_sum(table, idxp, ns, idx128, idx120, g_v, o_v, sem):
        wid = lax.axis_index("s") * 2 + lax.axis_index("c")
        zero = jnp.zeros((16,), jnp.float32)

        def zbody(r, carry):
            for f in range(Fc // 16):
                o_v[r, pl.ds(16 * f, 16)] = zero
            return carry

        lax.fori_loop(0, 125, zbody, 0)
        pltpu.sync_copy(o_v.at[pl.ds(0, 125)], ns.at[pl.ds(wid * 125, 125)])

        for d in range(1, 7):
            C, R = C_D[d], R_D[d]
            idx_v = idx128 if C == 128 else idx120
            base_idx = SEG_OFF[d] + wid * P_D[d]
            row0 = DEG_STARTS[d] + wid * ROWS_PER_TILE
            nfull = ROWS_PER_TILE // R
            rem = ROWS_PER_TILE - nfull * R

            def do_chunk(c, rows_c, d=d, C=C, R=R, base_idx=base_idx, row0=row0,
                         idx_v=idx_v):
                pltpu.sync_copy(idxp.at[pl.ds(base_idx + c * C, C)], idx_v)
                pltpu.async_copy(table.at[idx_v], g_v.at[pl.ds(0, C)], sem).wait()

                def body(r, carry):
                    for f in range(Fc // 16):
                        sl = pl.ds(16 * f, 16)
                        acc = g_v[r * d, sl]
                        for j in range(1, d):
                            acc = acc + g_v[r * d + j, sl]
                        o_v[r, sl] = acc
                    return carry

                lax.fori_loop(0, rows_c, body, 0)
                pltpu.sync_copy(o_v.at[pl.ds(0, rows_c)],
                                ns.at[pl.ds(row0 + c * R, rows_c)])

            def chunk_body(c, carry, do_chunk=do_chunk, R=R):
                do_chunk(c, R)
                return carry

            lax.fori_loop(0, nfull, chunk_body, 0)
            if rem:
                do_chunk(nfull, rem)

    return gather_sum


_gather_sum_128 = _make_gather_sum(128)
_gather_sum_64 = _make_gather_sum(64)


BLK = 4000
NBLK = N // BLK


def _widx(b):
    return jnp.where(b == 0, 0, (b - 1) // 4 + 1)


def _gcmm_body(ns_ref, self_ref, wn_ref, ws_ref, bias_ref, gamma_ref, beta_ref,
               h_ref, so_ref, acc_ref):
    b = pl.program_id(0)
    x = (jnp.dot(ns_ref[...], wn_ref[0], preferred_element_type=jnp.float32)
         + jnp.dot(self_ref[...], ws_ref[0], preferred_element_type=jnp.float32)
         + bias_ref[0])
    h = jnp.tanh(x)
    h_ref[...] = h

    @pl.when(b == 0)
    def _():
        acc_ref[...] = jnp.zeros_like(acc_ref)

    acc_ref[0:1, :] += jnp.sum(h, axis=0, keepdims=True)
    acc_ref[1:2, :] += jnp.sum(h * h, axis=0, keepdims=True)

    @pl.when(b == pl.num_programs(0) - 1)
    def _():
        mean = acc_ref[0:1, :] * (1.0 / N)
        var = acc_ref[1:2, :] * (1.0 / N) - mean * mean
        scale = gamma_ref[...] * jax.lax.rsqrt(var + 1e-3)
        so_ref[0:1, :] = scale
        so_ref[1:2, :] = beta_ref[...] - scale * mean


def _graph_conv_mm(ns, selfrows, Wn, Ws, bsum, gamma, beta, Fc):
    return pl.pallas_call(
        _gcmm_body,
        grid=(NBLK,),
        in_specs=[
            pl.BlockSpec((BLK, Fc), lambda b: (b, 0)),
            pl.BlockSpec((BLK, Fc), lambda b: (b, 0)),
            pl.BlockSpec((1, Fc, 64), lambda b: (_widx(b), 0, 0)),
            pl.BlockSpec((1, Fc, 64), lambda b: (_widx(b), 0, 0)),
            pl.BlockSpec((1, 1, 64), lambda b: (_widx(b), 0, 0)),
            pl.BlockSpec((1, 64), lambda b: (0, 0)),
            pl.BlockSpec((1, 64), lambda b: (0, 0)),
        ],
        out_specs=[
            pl.BlockSpec((BLK, 64), lambda b: (b, 0)),
            pl.BlockSpec((2, 64), lambda b: (0, 0)),
        ],
        out_shape=[
            jax.ShapeDtypeStruct((N, 64), jnp.float32),
            jax.ShapeDtypeStruct((2, 64), jnp.float32),
        ],
        scratch_shapes=[pltpu.VMEM((2, 64), jnp.float32)],
    )(ns, selfrows, Wn, Ws, bsum.reshape(7, 1, 64), gamma.reshape(1, 64),
      beta.reshape(1, 64))


def _make_pool(Fc):
    mesh = plsc.VectorSubcoreMesh(core_axis_name="c", subcore_axis_name="s")

    @functools.partial(
        pl.kernel,
        out_type=jax.ShapeDtypeStruct((N, Fc), jnp.float32),
        mesh=mesh,
        compiler_params=pltpu.CompilerParams(use_tc_tiling_on_sc=False),
        scratch_types=[
            pltpu.VMEM((128,), jnp.int32),
            pltpu.VMEM((120,), jnp.int32),
            pltpu.VMEM((128, Fc), jnp.float32),
            pltpu.VMEM((128, Fc), jnp.float32),
            pltpu.VMEM((128, Fc), jnp.float32),
            pltpu.VMEM((2, Fc), jnp.float32),
            pltpu.SemaphoreType.DMA,
        ],
    )
    def pool(h, idxp, so, p, idx128, idx120, g_v, s_v, o_v, so_v, sem):
        wid = lax.axis_index("s") * 2 + lax.axis_index("c")
        pltpu.sync_copy(so, so_v)

        pltpu.sync_copy(h.at[pl.ds(wid * 125, 125)], s_v.at[pl.ds(0, 125)])

        def z0(r, carry):
            for f in range(Fc // 16):
                sl = pl.ds(16 * f, 16)
                o_v[r, sl] = s_v[r, sl] * so_v[0, sl] + so_v[1, sl]
            return carry

        lax.fori_loop(0, 125, z0, 0)
        pltpu.sync_copy(o_v.at[pl.ds(0, 125)], p.at[pl.ds(wid * 125, 125)])

        for d in range(1, 7):
            C, R = C_D[d], R_D[d]
            idx_v = idx128 if C == 128 else idx120
            base_idx = SEG_OFF[d] + wid * P_D[d]
            row0 = DEG_STARTS[d] + wid * ROWS_PER_TILE
            nfull = ROWS_PER_TILE // R
            rem = ROWS_PER_TILE - nfull * R

            def do_chunk(c, rows_c, d=d, C=C, R=R, base_idx=base_idx,
                         row0=row0, idx_v=idx_v):
                pltpu.sync_copy(idxp.at[pl.ds(base_idx + c * C, C)], idx_v)
                pltpu.async_copy(h.at[idx_v], g_v.at[pl.ds(0, C)], sem).wait()
                pltpu.sync_copy(h.at[pl.ds(row0 + c * R, rows_c)],
                                s_v.at[pl.ds(0, rows_c)])

                def body(r, carry):
                    for f in range(Fc // 16):
                        sl = pl.ds(16 * f, 16)
                        m = s_v[r, sl]
                        for j in range(d):
                            m = jnp.maximum(m, g_v[r * d + j, sl])
                        o_v[r, sl] = m * so_v[0, sl] + so_v[1, sl]
                    return carry

                lax.fori_loop(0, rows_c, body, 0)
                pltpu.sync_copy(o_v.at[pl.ds(0, rows_c)],
                                p.at[pl.ds(row0 + c * R, rows_c)])

            def chunk_body(c, carry, do_chunk=do_chunk, R=R):
                do_chunk(c, R)
                return carry

            lax.fori_loop(0, nfull, chunk_body, 0)
            if rem:
                do_chunk(nfull, rem)

    return pool


_pool_64 = _make_pool(64)


def _split_weights(W, b, Fc):
    wn = jnp.concatenate([jnp.zeros((1, Fc, 64), jnp.float32), W[0:11:2]], axis=0)
    ws = jnp.concatenate([W[12:13], W[1:12:2]], axis=0)
    bs = jnp.concatenate([b[12:13], b[0:11:2] + b[1:12:2]], axis=0)
    return wn, ws, bs


def _d1_body(x_ref, w_ref, b_ref, gamma_ref, beta_ref, h_ref, so_ref, acc_ref):
    b = pl.program_id(0)
    h = jnp.tanh(jnp.dot(x_ref[...], w_ref[...], preferred_element_type=jnp.float32)
                 + b_ref[...])
    h_ref[...] = h

    @pl.when(b == 0)
    def _():
        acc_ref[...] = jnp.zeros_like(acc_ref)

    acc_ref[0:1, :] += jnp.sum(h, axis=0, keepdims=True)
    acc_ref[1:2, :] += jnp.sum(h * h, axis=0, keepdims=True)

    @pl.when(b == pl.num_programs(0) - 1)
    def _():
        mean = acc_ref[0:1, :] * (1.0 / N)
        var = acc_ref[1:2, :] * (1.0 / N) - mean * mean
        scale = gamma_ref[...] * jax.lax.rsqrt(var + 1e-3)
        so_ref[0:1, :] = scale
        so_ref[1:2, :] = beta_ref[...] - scale * mean


def _dense1(x, W, bias, gamma, beta):
    return pl.pallas_call(
        _d1_body,
        grid=(NBLK,),
        in_specs=[
            pl.BlockSpec((BLK, 64), lambda b: (b, 0)),
            pl.BlockSpec((64, 128), lambda b: (0, 0)),
            pl.BlockSpec((1, 128), lambda b: (0, 0)),
            pl.BlockSpec((1, 128), lambda b: (0, 0)),
            pl.BlockSpec((1, 128), lambda b: (0, 0)),
        ],
        out_specs=[
            pl.BlockSpec((BLK, 128), lambda b: (b, 0)),
            pl.BlockSpec((2, 128), lambda b: (0, 0)),
        ],
        out_shape=[
            jax.ShapeDtypeStruct((N, 128), jnp.float32),
            jax.ShapeDtypeStruct((2, 128), jnp.float32),
        ],
        scratch_shapes=[pltpu.VMEM((2, 128), jnp.float32)],
    )(x, W, bias.reshape(1, 128), gamma.reshape(1, 128), beta.reshape(1, 128))


SEGB = 2048
DUMMY = BATCH
_SUM_SPAN = 3200
_FULL_CHUNKS = N // 128
_TAIL = N - _FULL_CHUNKS * 128


def _make_segment():
    mesh = plsc.VectorSubcoreMesh(core_axis_name="c", subcore_axis_name="s")

    @functools.partial(
        pl.kernel,
        out_type=[
            jax.ShapeDtypeStruct((2, SEGB, 128), jnp.float32),
            jax.ShapeDtypeStruct((8, SEGB, 128), jnp.float32),
        ],
        mesh=mesh,
        compiler_params=pltpu.CompilerParams(use_tc_tiling_on_sc=False),
        scratch_types=[
            pltpu.VMEM((128, 128), jnp.float32),
            pltpu.VMEM((128, 32), jnp.float32),
            pltpu.VMEM((128,), jnp.int32),
            pltpu.VMEM((SEGB, 32), jnp.float32),
            pltpu.VMEM((2, 128), jnp.float32),
            pltpu.VMEM_SHARED((SEGB, 128), jnp.float32),
            pltpu.SemaphoreType.DMA,
        ],
    )
    def segment(h3, mem, so, ssum_out, mx_out,
                x_v, xb_v, mem_v, acc_v, so_v, shared, sem):
        cid = lax.axis_index("c")
        sid = lax.axis_index("s")
        wid = sid * 2 + cid
        pltpu.sync_copy(so, so_v)

        zero = jnp.zeros((16,), jnp.float32)

        def zx(r, carry):
            for f in range(8):
                x_v[r, pl.ds(16 * f, 16)] = zero
            return carry

        lax.fori_loop(0, 128, zx, 0)
        pltpu.sync_copy(x_v, shared.at[pl.ds(sid * 128, 128)])
        plsc.subcore_barrier()

        def sum_chunk(base, nrows):
            pltpu.sync_copy(mem.at[pl.ds(base, nrows)], mem_v.at[pl.ds(0, nrows)])
            if nrows < 128:
                dummy = jnp.full((16,), DUMMY, jnp.int32)
                for k in range((128 - nrows) // 16):
                    mem_v[pl.ds(nrows + 16 * k, 16)] = dummy
            pltpu.sync_copy(h3.at[pl.ds(base, nrows)], x_v.at[pl.ds(0, nrows)])

            def abody(r, carry):
                for f in range(8):
                    sl = pl.ds(16 * f, 16)
                    x_v[r, sl] = x_v[r, sl] * so_v[0, sl] + so_v[1, sl]
                return carry

            lax.fori_loop(0, 128, abody, 0)
            pltpu.sync_copy(x_v, shared.at[mem_v], add=True)

        @pl.when(wid < 31)
        def _():
            def cbody(c, carry):
                sum_chunk(wid * _SUM_SPAN + c * 128, 128)
                return carry

            lax.fori_loop(0, 25, cbody, 0)

        @pl.when(wid == 31)
        def _():
            def cbody(c, carry):
                sum_chunk(31 * _SUM_SPAN + c * 128, 128)
                return carry

            lax.fori_loop(0, 6, cbody, 0)
            sum_chunk(_FULL_CHUNKS * 128, _TAIL)

        plsc.subcore_barrier()
        pltpu.sync_copy(shared.at[pl.ds(sid * 128, 128)],
                        ssum_out.at[cid, pl.ds(sid * 128, 128)])

        rg = wid // 4
        fg = wid % 4
        fof = fg * 32
        ninf = jnp.full((16,), -jnp.inf, jnp.float32)

        def zacc(row, carry):
            acc_v[row, pl.ds(0, 16)] = ninf
            acc_v[row, pl.ds(16, 16)] = ninf
            return carry

        lax.fori_loop(0, SEGB, zacc, 0)

        def max_chunk(base, nrows):
            pltpu.sync_copy(mem.at[pl.ds(base, nrows)], mem_v.at[pl.ds(0, nrows)])
            pltpu.sync_copy(h3.at[pl.ds(base, nrows), pl.ds(fof, 32)],
                            xb_v.at[pl.ds(0, nrows)])

            def mbody(g, carry):
                mv = mem_v[pl.ds(g * 16, 16)]
                for k in range(16):
                    m = mv[k]
                    r = g * 16 + k
                    for f2 in range(2):
                        sl = pl.ds(16 * f2, 16)
                        acc_v[m, sl] = jnp.maximum(acc_v[m, sl], xb_v[r, sl])
                return carry

            lax.fori_loop(0, nrows // 16, mbody, 0)

        @pl.when(rg < 7)
        def _():
            def cbody(c, carry):
                max_chunk((rg * 98 + c) * 128, 128)
                return carry

            lax.fori_loop(0, 98, cbody, 0)

        @pl.when(rg == 7)
        def _():
            def cbody(c, carry):
                max_chunk((686 + c) * 128, 128)
                return carry

            lax.fori_loop(0, 95, cbody, 0)
            max_chunk(_FULL_CHUNKS * 128, _TAIL)

        def affb(row, carry):
            for f2 in range(2):
                sl = pl.ds(16 * f2, 16)
                fsl = pl.ds(fof + 16 * f2, 16)
                acc_v[row, sl] = acc_v[row, sl] * so_v[0, fsl] + so_v[1, fsl]
            return carry

        lax.fori_loop(0, SEGB, affb, 0)
        pltpu.sync_copy(acc_v, mx_out.at[rg, pl.ds(0, SEGB), pl.ds(fof, 32)])

    return segment


_segment = _make_segment()


def _final_body(ssum_ref, mx_ref, d2w_ref, d2b_ref, d3w_ref, d3b_ref, out_ref):
    s = ssum_ref[0] + ssum_ref[1]
    m = jnp.max(mx_ref[...], axis=0)
    g = jnp.tanh(jnp.concatenate([s[0:BATCH], m[0:BATCH]], axis=1))
    z = jax.nn.sigmoid(jnp.dot(g, d2w_ref[...], preferred_element_type=jnp.float32)
                       + d2b_ref[...])
    out_ref[...] = (jnp.dot(z, d3w_ref[...], preferred_element_type=jnp.float32)
                    + d3b_ref[...])


def kernel(atom_features, degree_slice, membership, deg_adj_1, deg_adj_2, deg_adj_3, deg_adj_4, deg_adj_5, deg_adj_6, gc1_W, gc1_b, gc2_W, gc2_b, bn1_gamma, bn1_beta, bn3_gamma, bn3_beta, d1_W, d1_b, d2_W, d2_b, d3_W, d3_b):
    deg_adj_lists = [deg_adj_1, deg_adj_2, deg_adj_3, deg_adj_4, deg_adj_5, deg_adj_6]
    idxp = _build_padded_idx(deg_adj_lists)

    wn1, ws1, bs1 = _split_weights(gc1_W, gc1_b, F)
    wn2, ws2, bs2 = _split_weights(gc2_W, gc2_b, 64)

    ns1 = _gather_sum_128(atom_features, idxp)
    h1, so1 = _graph_conv_mm(ns1, atom_features, wn1, ws1, bs1, bn1_gamma, bn1_beta, F)
    p1 = _pool_64(h1, idxp, so1)
    ns2 = _gather_sum_64(p1, idxp)
    h2, so2 = _graph_conv_mm(ns2, p1, wn2, ws2, bs2, bn1_gamma, bn1_beta, 64)
    p2 = _pool_64(h2, idxp, so2)
    h3, so3 = _dense1(p2, d1_W, d1_b, bn3_gamma, bn3_beta)
    ssum, mx = _segment(h3, membership, so3)
    out = pl.pallas_call(
        _final_body,
        out_shape=jax.ShapeDtypeStruct((BATCH, 1), jnp.float32),
    )(ssum, mx, d2_W, d2_b.reshape(1, 64), d3_W, d3_b.reshape(1, 1))
    return out


# speedup vs baseline: 2.6826x; 1.3608x over previous
import functools

import jax
import jax.numpy as jnp
from jax import lax
from jax.experimental import pallas as pl
from jax.experimental.pallas import tpu as pltpu
from jax.experimental.pallas import tpu_sc as plsc

MAX_DEG = 6
N = 100000
F = 128
BATCH = 2000
DEG_SIZES = [4000, 16000, 16000, 16000, 16000, 16000, 16000]
DEG_STARTS = [0, 4000, 20000, 36000, 52000, 68000, 84000]
NW = 32
ROWS_PER_TILE = 500

C_D = {1: 128, 2: 128, 3: 120, 4: 128, 5: 120, 6: 120}
R_D = {d: C_D[d] // d for d in C_D}
NCHUNK = {d: -(-ROWS_PER_TILE // R_D[d]) for d in C_D}
P_D = {d: NCHUNK[d] * C_D[d] for d in C_D}
SEG_OFF = {}
_off = 0
for _d in range(1, 7):
    SEG_OFF[_d] = _off
    _off += NW * P_D[_d]
IDX_TOTAL = _off


def _build_padded_idx(adj_lists):
    segs = []
    for d in range(1, 7):
        a = adj_lists[d - 1].reshape(NW, ROWS_PER_TILE * d)
        a = jnp.pad(a, ((0, 0), (0, P_D[d] - ROWS_PER_TILE * d)))
        segs.append(a.reshape(-1))
    return jnp.concatenate(segs)


def _pipelined_degs(table, idxp, wid, idxall, gbufs, gsems, compute_chunk,
                    prefetch_self=None):
    g0, g1 = gbufs
    sem0, sem1 = gsems

    for d in range(1, 7):
        C, R = C_D[d], R_D[d]
        base_idx = SEG_OFF[d] + wid * P_D[d]
        nfull = ROWS_PER_TILE // R
        rem = ROWS_PER_TILE - nfull * R
        nchunk = nfull + (1 if rem else 0)
        pltpu.sync_copy(idxp.at[pl.ds(base_idx, P_D[d])],
                        idxall.at[pl.ds(0, P_D[d])])

        def start(c, buf, sem, C=C):
            pltpu.async_copy(table.at[idxall.at[pl.ds(c * C, C)]],
                             buf.at[pl.ds(0, C)], sem)
            if prefetch_self is not None:
                prefetch_self(d, c, buf is g1)

        def wait(buf, sem, C=C):
            pltpu.make_async_copy(table.at[idxall.at[pl.ds(0, C)]],
                                  buf.at[pl.ds(0, C)], sem).wait()

        pairs = (nchunk - 1) // 2
        tail = nchunk - 2 * pairs
        start(0, g0, sem0)
        if pairs:
            def pairbody(i, carry, d=d, R=R):
                c0 = 2 * i
                start(c0 + 1, g1, sem1)
                wait(g0, sem0)
                compute_chunk(d, c0, 0, R)
                start(c0 + 2, g0, sem0)
                wait(g1, sem1)
                compute_chunk(d, c0 + 1, 1, R)
                return carry

            lax.fori_loop(0, pairs, pairbody, 0)
        t0 = 2 * pairs
        last_rows = rem if rem else R
        if tail == 2:
            start(t0 + 1, g1, sem1)
            wait(g0, sem0)
            compute_chunk(d, t0, 0, R)
            wait(g1, sem1)
            compute_chunk(d, t0 + 1, 1, last_rows)
        else:
            wait(g0, sem0)
            compute_chunk(d, t0, 0, last_rows)


def _make_gather_sum(Fc):
    mesh = plsc.VectorSubcoreMesh(core_axis_name="c", subcore_axis_name="s")

    @functools.partial(
        pl.kernel,
        out_type=jax.ShapeDtypeStruct((N, Fc), jnp.float32),
        mesh=mesh,
        compiler_params=pltpu.CompilerParams(use_tc_tiling_on_sc=False),
        scratch_types=[
            pltpu.VMEM((3008,), jnp.int32),
            pltpu.VMEM((128, Fc), jnp.float32),
            pltpu.VMEM((128, Fc), jnp.float32),
            pltpu.VMEM((128, Fc), jnp.float32),
            pltpu.SemaphoreType.DMA,
            pltpu.SemaphoreType.DMA,
        ],
    )
    def gather_sum(table, idxp, ns, idxall, g0, g1, o_v, sem0, sem1):
        wid = lax.axis_index("s") * 2 + lax.axis_index("c")
        zero = jnp.zeros((16,), jnp.float32)

        def zbody(r, carry):
            for f in range(Fc // 16):
                o_v[r, pl.ds(16 * f, 16)] = zero
            return carry

        lax.fori_loop(0, 125, zbody, 0)
        pltpu.sync_copy(o_v.at[pl.ds(0, 125)], ns.at[pl.ds(wid * 125, 125)])

        def compute_chunk(d, c, gb, rows_c):
            g_v = g1 if gb else g0
            row0 = DEG_STARTS[d] + wid * ROWS_PER_TILE

            def body(r, carry):
                for f in range(Fc // 16):
                    sl = pl.ds(16 * f, 16)
                    acc = g_v[r * d, sl]
                    for j in range(1, d):
                        acc = acc + g_v[r * d + j, sl]
                    o_v[r, sl] = acc
                return carry

            lax.fori_loop(0, rows_c, body, 0)
            pltpu.sync_copy(o_v.at[pl.ds(0, rows_c)],
                            ns.at[pl.ds(row0 + c * R_D[d], rows_c)])

        _pipelined_degs(table, idxp, wid, idxall, (g0, g1), (sem0, sem1),
                        compute_chunk)

    return gather_sum


_gather_sum_128 = _make_gather_sum(128)
_gather_sum_64 = _make_gather_sum(64)


BLK = 4000
NBLK = N // BLK


def _widx(b):
    return jnp.where(b == 0, 0, (b - 1) // 4 + 1)


def _gcmm_body(ns_ref, self_ref, wn_ref, ws_ref, bias_ref, gamma_ref, beta_ref,
               h_ref, so_ref, acc_ref):
    b = pl.program_id(0)
    x = (jnp.dot(ns_ref[...], wn_ref[0], preferred_element_type=jnp.float32)
         + jnp.dot(self_ref[...], ws_ref[0], preferred_element_type=jnp.float32)
         + bias_ref[0])
    h = jnp.tanh(x)
    h_ref[...] = h

    @pl.when(b == 0)
    def _():
        acc_ref[...] = jnp.zeros_like(acc_ref)

    acc_ref[0:1, :] += jnp.sum(h, axis=0, keepdims=True)
    acc_ref[1:2, :] += jnp.sum(h * h, axis=0, keepdims=True)

    @pl.when(b == pl.num_programs(0) - 1)
    def _():
        mean = acc_ref[0:1, :] * (1.0 / N)
        var = acc_ref[1:2, :] * (1.0 / N) - mean * mean
        scale = gamma_ref[...] * jax.lax.rsqrt(var + 1e-3)
        so_ref[0:1, :] = scale
        so_ref[1:2, :] = beta_ref[...] - scale * mean


def _graph_conv_mm(ns, selfrows, Wn, Ws, bsum, gamma, beta, Fc):
    return pl.pallas_call(
        _gcmm_body,
        grid=(NBLK,),
        in_specs=[
            pl.BlockSpec((BLK, Fc), lambda b: (b, 0)),
            pl.BlockSpec((BLK, Fc), lambda b: (b, 0)),
            pl.BlockSpec((1, Fc, 64), lambda b: (_widx(b), 0, 0)),
            pl.BlockSpec((1, Fc, 64), lambda b: (_widx(b), 0, 0)),
            pl.BlockSpec((1, 1, 64), lambda b: (_widx(b), 0, 0)),
            pl.BlockSpec((1, 64), lambda b: (0, 0)),
            pl.BlockSpec((1, 64), lambda b: (0, 0)),
        ],
        out_specs=[
            pl.BlockSpec((BLK, 64), lambda b: (b, 0)),
            pl.BlockSpec((2, 64), lambda b: (0, 0)),
        ],
        out_shape=[
            jax.ShapeDtypeStruct((N, 64), jnp.float32),
            jax.ShapeDtypeStruct((2, 64), jnp.float32),
        ],
        scratch_shapes=[pltpu.VMEM((2, 64), jnp.float32)],
    )(ns, selfrows, Wn, Ws, bsum.reshape(7, 1, 64), gamma.reshape(1, 64),
      beta.reshape(1, 64))


def _make_pool(Fc):
    mesh = plsc.VectorSubcoreMesh(core_axis_name="c", subcore_axis_name="s")

    @functools.partial(
        pl.kernel,
        out_type=jax.ShapeDtypeStruct((N, Fc), jnp.float32),
        mesh=mesh,
        compiler_params=pltpu.CompilerParams(use_tc_tiling_on_sc=False),
        scratch_types=[
            pltpu.VMEM((3008,), jnp.int32),
            pltpu.VMEM((128, Fc), jnp.float32),
            pltpu.VMEM((128, Fc), jnp.float32),
            pltpu.VMEM((128, Fc), jnp.float32),
            pltpu.VMEM((128, Fc), jnp.float32),
            pltpu.VMEM((128, Fc), jnp.float32),
            pltpu.VMEM((2, Fc), jnp.float32),
            pltpu.SemaphoreType.DMA,
            pltpu.SemaphoreType.DMA,
            pltpu.SemaphoreType.DMA,
            pltpu.SemaphoreType.DMA,
        ],
    )
    def pool(h, idxp, so, p, idxall, g0, g1, s0, s1, o_v, so_v,
             sem0, sem1, ss0, ss1):
        wid = lax.axis_index("s") * 2 + lax.axis_index("c")
        pltpu.sync_copy(so, so_v)

        pltpu.sync_copy(h.at[pl.ds(wid * 125, 125)], s0.at[pl.ds(0, 125)])

        def z0(r, carry):
            for f in range(Fc // 16):
                sl = pl.ds(16 * f, 16)
                o_v[r, sl] = s0[r, sl] * so_v[0, sl] + so_v[1, sl]
            return carry

        lax.fori_loop(0, 125, z0, 0)
        pltpu.sync_copy(o_v.at[pl.ds(0, 125)], p.at[pl.ds(wid * 125, 125)])

        def prefetch_self(d, c, sb):
            row0 = DEG_STARTS[d] + wid * ROWS_PER_TILE
            R = R_D[d]
            pltpu.async_copy(h.at[pl.ds(row0 + c * R, R)],
                             (s1 if sb else s0).at[pl.ds(0, R)],
                             ss1 if sb else ss0)

        def compute_chunk(d, c, gb, rows_c):
            g_v = g0 if gb == 0 else g1
            s_v = s0 if gb == 0 else s1
            ssem = ss0 if gb == 0 else ss1
            R = R_D[d]
            row0 = DEG_STARTS[d] + wid * ROWS_PER_TILE
            pltpu.make_async_copy(h.at[pl.ds(row0, R)],
                                  s_v.at[pl.ds(0, R)], ssem).wait()

            def body(r, carry):
                for f in range(Fc // 16):
                    sl = pl.ds(16 * f, 16)
                    m = s_v[r, sl]
                    for j in range(d):
                        m = jnp.maximum(m, g_v[r * d + j, sl])
                    o_v[r, sl] = m * so_v[0, sl] + so_v[1, sl]
                return carry

            lax.fori_loop(0, rows_c, body, 0)
            pltpu.sync_copy(o_v.at[pl.ds(0, rows_c)],
                            p.at[pl.ds(row0 + c * R, rows_c)])

        _pipelined_degs(h, idxp, wid, idxall, (g0, g1), (sem0, sem1),
                        compute_chunk, prefetch_self=prefetch_self)

    return pool


_pool_64 = _make_pool(64)


def _split_weights(W, b, Fc):
    wn = jnp.concatenate([jnp.zeros((1, Fc, 64), jnp.float32), W[0:11:2]], axis=0)
    ws = jnp.concatenate([W[12:13], W[1:12:2]], axis=0)
    bs = jnp.concatenate([b[12:13], b[0:11:2] + b[1:12:2]], axis=0)
    return wn, ws, bs


def _d1_body(x_ref, w_ref, b_ref, gamma_ref, beta_ref, h_ref, so_ref, acc_ref):
    b = pl.program_id(0)
    h = jnp.tanh(jnp.dot(x_ref[...], w_ref[...], preferred_element_type=jnp.float32)
                 + b_ref[...])
    h_ref[...] = h

    @pl.when(b == 0)
    def _():
        acc_ref[...] = jnp.zeros_like(acc_ref)

    acc_ref[0:1, :] += jnp.sum(h, axis=0, keepdims=True)
    acc_ref[1:2, :] += jnp.sum(h * h, axis=0, keepdims=True)

    @pl.when(b == pl.num_programs(0) - 1)
    def _():
        mean = acc_ref[0:1, :] * (1.0 / N)
        var = acc_ref[1:2, :] * (1.0 / N) - mean * mean
        scale = gamma_ref[...] * jax.lax.rsqrt(var + 1e-3)
        so_ref[0:1, :] = scale
        so_ref[1:2, :] = beta_ref[...] - scale * mean


def _dense1(x, W, bias, gamma, beta):
    return pl.pallas_call(
        _d1_body,
        grid=(NBLK,),
        in_specs=[
            pl.BlockSpec((BLK, 64), lambda b: (b, 0)),
            pl.BlockSpec((64, 128), lambda b: (0, 0)),
            pl.BlockSpec((1, 128), lambda b: (0, 0)),
            pl.BlockSpec((1, 128), lambda b: (0, 0)),
            pl.BlockSpec((1, 128), lambda b: (0, 0)),
        ],
        out_specs=[
            pl.BlockSpec((BLK, 128), lambda b: (b, 0)),
            pl.BlockSpec((2, 128), lambda b: (0, 0)),
        ],
        out_shape=[
            jax.ShapeDtypeStruct((N, 128), jnp.float32),
            jax.ShapeDtypeStruct((2, 128), jnp.float32),
        ],
        scratch_shapes=[pltpu.VMEM((2, 128), jnp.float32)],
    )(x, W, bias.reshape(1, 128), gamma.reshape(1, 128), beta.reshape(1, 128))


SEGB = 2048
DUMMY = BATCH
_SUM_SPAN = 3200
_FULL_CHUNKS = N // 128
_TAIL = N - _FULL_CHUNKS * 128


def _make_segment():
    mesh = plsc.VectorSubcoreMesh(core_axis_name="c", subcore_axis_name="s")

    @functools.partial(
        pl.kernel,
        out_type=[
            jax.ShapeDtypeStruct((2, SEGB, 128), jnp.float32),
            jax.ShapeDtypeStruct((8, SEGB, 128), jnp.float32),
        ],
        mesh=mesh,
        compiler_params=pltpu.CompilerParams(use_tc_tiling_on_sc=False),
        scratch_types=[
            pltpu.VMEM((128, 128), jnp.float32),
            pltpu.VMEM((128, 128), jnp.float32),
            pltpu.VMEM((128, 32), jnp.float32),
            pltpu.VMEM((128, 32), jnp.float32),
            pltpu.VMEM((128,), jnp.int32),
            pltpu.VMEM((128,), jnp.int32),
            pltpu.VMEM((SEGB, 32), jnp.float32),
            pltpu.VMEM((2, 128), jnp.float32),
            pltpu.VMEM_SHARED((SEGB, 128), jnp.float32),
            pltpu.SemaphoreType.DMA,
            pltpu.SemaphoreType.DMA,
            pltpu.SemaphoreType.DMA,
            pltpu.SemaphoreType.DMA,
        ],
    )
    def segment(h3, mem, so, ssum_out, mx_out,
                x0, x1, xb0, xb1, m0, m1, acc_v, so_v, shared,
                sx0, sx1, sm0, sm1):
        cid = lax.axis_index("c")
        sid = lax.axis_index("s")
        wid = sid * 2 + cid
        pltpu.sync_copy(so, so_v)

        zero = jnp.zeros((16,), jnp.float32)

        def zx(r, carry):
            for f in range(8):
                x0[r, pl.ds(16 * f, 16)] = zero
            return carry

        lax.fori_loop(0, 128, zx, 0)
        pltpu.sync_copy(x0, shared.at[pl.ds(sid * 128, 128)])
        plsc.subcore_barrier()

        xbufs = (x0, x1)
        mbufs = (m0, m1)
        sxs = (sx0, sx1)
        sms = (sm0, sm1)

        def startA(base, b, nrows):
            pltpu.async_copy(mem.at[pl.ds(base, nrows)],
                             mbufs[b].at[pl.ds(0, nrows)], sms[b])
            pltpu.async_copy(h3.at[pl.ds(base, nrows)],
                             xbufs[b].at[pl.ds(0, nrows)], sxs[b])

        def computeA(b, nrows):
            x_v, mem_v = xbufs[b], mbufs[b]
            pltpu.make_async_copy(mem.at[pl.ds(0, nrows)],
                                  mem_v.at[pl.ds(0, nrows)], sms[b]).wait()
            pltpu.make_async_copy(h3.at[pl.ds(0, nrows)],
                                  x_v.at[pl.ds(0, nrows)], sxs[b]).wait()
            if nrows < 128:
                dummy = jnp.full((16,), DUMMY, jnp.int32)
                for k in range((128 - nrows) // 16):
                    mem_v[pl.ds(nrows + 16 * k, 16)] = dummy

            def abody(r, carry):
                for f in range(8):
                    sl = pl.ds(16 * f, 16)
                    x_v[r, sl] = x_v[r, sl] * so_v[0, sl] + so_v[1, sl]
                return carry

            lax.fori_loop(0, 128, abody, 0)
            pltpu.sync_copy(x_v, shared.at[mem_v], add=True)

        def pipeline2(nchunk, last_rows, start_fn, compute_fn):
            size = lambda c: last_rows if c == nchunk - 1 else 128
            if nchunk == 1:
                start_fn(0, 0, last_rows)
                compute_fn(0, last_rows)
                return
            pairs = ((nchunk - 1) // 2 if last_rows == 128
                     else max(0, (nchunk - 2) // 2))
            start_fn(0, 0, 128)
            if pairs:
                def pb(i, carry):
                    c0 = 2 * i
                    start_fn(c0 + 1, 1, 128)
                    compute_fn(0, 128)
                    start_fn(c0 + 2, 0, 128)
                    compute_fn(1, 128)
                    return carry

                lax.fori_loop(0, pairs, pb, 0)
            t0 = 2 * pairs
            remc = nchunk - t0
            if remc == 1:
                compute_fn(0, size(t0))
            elif remc == 2:
                start_fn(t0 + 1, 1, size(t0 + 1))
                compute_fn(0, 128)
                compute_fn(1, size(t0 + 1))
            else:
                start_fn(t0 + 1, 1, 128)
                compute_fn(0, 128)
                start_fn(t0 + 2, 0, size(t0 + 2))
                compute_fn(1, 128)
                compute_fn(0, size(t0 + 2))

        def phaseA(nchunk, base0, last_rows):
            pipeline2(nchunk, last_rows,
                      lambda c, b, nr: startA(base0 + c * 128, b, nr),
                      computeA)

        @pl.when(wid < 31)
        def _():
            phaseA(25, wid * _SUM_SPAN, 128)

        @pl.when(wid == 31)
        def _():
            phaseA(7, 31 * _SUM_SPAN, _TAIL)

        plsc.subcore_barrier()
        pltpu.sync_copy(shared.at[pl.ds(sid * 128, 128)],
                        ssum_out.at[cid, pl.ds(sid * 128, 128)])

        rg = wid // 4
        fg = wid % 4
        fof = fg * 32
        ninf = jnp.full((16,), -jnp.inf, jnp.float32)

        def zacc(row, carry):
            acc_v[row, pl.ds(0, 16)] = ninf
            acc_v[row, pl.ds(16, 16)] = ninf
            return carry

        lax.fori_loop(0, SEGB, zacc, 0)

        xbbufs = (xb0, xb1)

        def startB(base, b, nrows):
            pltpu.async_copy(mem.at[pl.ds(base, nrows)],
                             mbufs[b].at[pl.ds(0, nrows)], sms[b])
            pltpu.async_copy(h3.at[pl.ds(base, nrows), pl.ds(fof, 32)],
                             xbbufs[b].at[pl.ds(0, nrows)], sxs[b])

        def computeB(b, nrows):
            xb_v, mem_v = xbbufs[b], mbufs[b]
            pltpu.make_async_copy(mem.at[pl.ds(0, nrows)],
                                  mem_v.at[pl.ds(0, nrows)], sms[b]).wait()
            pltpu.make_async_copy(h3.at[pl.ds(0, nrows), pl.ds(0, 32)],
                                  xb_v.at[pl.ds(0, nrows)], sxs[b]).wait()

            def mbody(g, carry):
                mv = mem_v[pl.ds(g * 16, 16)]
                for k in range(16):
                    m = mv[k]
                    r = g * 16 + k
                    for f2 in range(2):
                        sl = pl.ds(16 * f2, 16)
                        acc_v[m, sl] = jnp.maximum(acc_v[m, sl], xb_v[r, sl])
                return carry

            lax.fori_loop(0, nrows // 16, mbody, 0)

        def phaseB(nchunk, coff, last_rows):
            pipeline2(nchunk, last_rows,
                      lambda c, b, nr: startB((coff + c) * 128, b, nr),
                      computeB)

        @pl.when(rg < 7)
        def _():
            phaseB(98, rg * 98, 128)

        @pl.when(rg == 7)
        def _():
            phaseB(96, 686, _TAIL)

        def affb(row, carry):
            for f2 in range(2):
                sl = pl.ds(16 * f2, 16)
                fsl = pl.ds(fof + 16 * f2, 16)
                acc_v[row, sl] = acc_v[row, sl] * so_v[0, fsl] + so_v[1, fsl]
            return carry

        lax.fori_loop(0, SEGB, affb, 0)
        pltpu.sync_copy(acc_v, mx_out.at[rg, pl.ds(0, SEGB), pl.ds(fof, 32)])

    return segment


_segment = _make_segment()


def _final_body(ssum_ref, mx_ref, d2w_ref, d2b_ref, d3w_ref, d3b_ref, out_ref):
    s = ssum_ref[0] + ssum_ref[1]
    m = jnp.max(mx_ref[...], axis=0)
    g = jnp.tanh(jnp.concatenate([s[0:BATCH], m[0:BATCH]], axis=1))
    z = jax.nn.sigmoid(jnp.dot(g, d2w_ref[...], preferred_element_type=jnp.float32)
                       + d2b_ref[...])
    out_ref[...] = (jnp.dot(z, d3w_ref[...], preferred_element_type=jnp.float32)
                    + d3b_ref[...])


def kernel(atom_features, degree_slice, membership, deg_adj_1, deg_adj_2, deg_adj_3, deg_adj_4, deg_adj_5, deg_adj_6, gc1_W, gc1_b, gc2_W, gc2_b, bn1_gamma, bn1_beta, bn3_gamma, bn3_beta, d1_W, d1_b, d2_W, d2_b, d3_W, d3_b):
    deg_adj_lists = [deg_adj_1, deg_adj_2, deg_adj_3, deg_adj_4, deg_adj_5, deg_adj_6]
    idxp = _build_padded_idx(deg_adj_lists)

    wn1, ws1, bs1 = _split_weights(gc1_W, gc1_b, F)
    wn2, ws2, bs2 = _split_weights(gc2_W, gc2_b, 64)

    ns1 = _gather_sum_128(atom_features, idxp)
    h1, so1 = _graph_conv_mm(ns1, atom_features, wn1, ws1, bs1, bn1_gamma, bn1_beta, F)
    p1 = _pool_64(h1, idxp, so1)
    ns2 = _gather_sum_64(p1, idxp)
    h2, so2 = _graph_conv_mm(ns2, p1, wn2, ws2, bs2, bn1_gamma, bn1_beta, 64)
    p2 = _pool_64(h2, idxp, so2)
    h3, so3 = _dense1(p2, d1_W, d1_b, bn3_gamma, bn3_beta)
    ssum, mx = _segment(h3, membership, so3)
    out = pl.pallas_call(
        _final_body,
        out_shape=jax.ShapeDtypeStruct((BATCH, 1), jnp.float32),
    )(ssum, mx, d2_W, d2_b.reshape(1, 64), d3_W, d3_b.reshape(1, 1))
    return out


# speedup vs baseline: 3.0074x; 1.1211x over previous
import functools

import jax
import jax.numpy as jnp
from jax import lax
from jax.experimental import pallas as pl
from jax.experimental.pallas import tpu as pltpu
from jax.experimental.pallas import tpu_sc as plsc

MAX_DEG = 6
N = 100000
F = 128
BATCH = 2000
DEG_SIZES = [4000, 16000, 16000, 16000, 16000, 16000, 16000]
DEG_STARTS = [0, 4000, 20000, 36000, 52000, 68000, 84000]
NW = 32
ROWS_PER_TILE = 500

C_D = {1: 128, 2: 128, 3: 120, 4: 128, 5: 120, 6: 120}
R_D = {d: C_D[d] // d for d in C_D}
NCHUNK = {d: -(-ROWS_PER_TILE // R_D[d]) for d in C_D}
P_D = {d: NCHUNK[d] * C_D[d] for d in C_D}
SEG_OFF = {}
_off = 0
for _d in range(1, 7):
    SEG_OFF[_d] = _off
    _off += NW * P_D[_d]
IDX_TOTAL = _off


def _build_padded_idx(adj_lists):
    segs = []
    for d in range(1, 7):
        a = adj_lists[d - 1].reshape(NW, ROWS_PER_TILE * d)
        a = jnp.pad(a, ((0, 0), (0, P_D[d] - ROWS_PER_TILE * d)))
        segs.append(a.reshape(-1))
    return jnp.concatenate(segs)


def _pipelined_degs(table, idxp, wid, idxall, gbufs, gsems, compute_chunk,
                    prefetch_self=None):
    g0, g1 = gbufs
    sem0, sem1 = gsems

    for d in range(1, 7):
        C, R = C_D[d], R_D[d]
        base_idx = SEG_OFF[d] + wid * P_D[d]
        nfull = ROWS_PER_TILE // R
        rem = ROWS_PER_TILE - nfull * R
        nchunk = nfull + (1 if rem else 0)
        pltpu.sync_copy(idxp.at[pl.ds(base_idx, P_D[d])],
                        idxall.at[pl.ds(0, P_D[d])])

        def start(c, buf, sem, C=C):
            pltpu.async_copy(table.at[idxall.at[pl.ds(c * C, C)]],
                             buf.at[pl.ds(0, C)], sem)
            if prefetch_self is not None:
                prefetch_self(d, c, buf is g1)

        def wait(buf, sem, C=C):
            pltpu.make_async_copy(table.at[idxall.at[pl.ds(0, C)]],
                                  buf.at[pl.ds(0, C)], sem).wait()

        pairs = (nchunk - 1) // 2
        tail = nchunk - 2 * pairs
        start(0, g0, sem0)
        if pairs:
            def pairbody(i, carry, d=d, R=R):
                c0 = 2 * i
                start(c0 + 1, g1, sem1)
                wait(g0, sem0)
                compute_chunk(d, c0, 0, R)
                start(c0 + 2, g0, sem0)
                wait(g1, sem1)
                compute_chunk(d, c0 + 1, 1, R)
                return carry

            lax.fori_loop(0, pairs, pairbody, 0)
        t0 = 2 * pairs
        last_rows = rem if rem else R
        if tail == 2:
            start(t0 + 1, g1, sem1)
            wait(g0, sem0)
            compute_chunk(d, t0, 0, R)
            wait(g1, sem1)
            compute_chunk(d, t0 + 1, 1, last_rows)
        else:
            wait(g0, sem0)
            compute_chunk(d, t0, 0, last_rows)


def _make_gather_sum(Fc):
    mesh = plsc.VectorSubcoreMesh(core_axis_name="c", subcore_axis_name="s")

    @functools.partial(
        pl.kernel,
        out_type=jax.ShapeDtypeStruct((N, Fc), jnp.float32),
        mesh=mesh,
        compiler_params=pltpu.CompilerParams(use_tc_tiling_on_sc=False),
        scratch_types=[
            pltpu.VMEM((3008,), jnp.int32),
            pltpu.VMEM((128, Fc), jnp.float32),
            pltpu.VMEM((128, Fc), jnp.float32),
            pltpu.VMEM((128, Fc), jnp.float32),
            pltpu.SemaphoreType.DMA,
            pltpu.SemaphoreType.DMA,
        ],
    )
    def gather_sum(table, idxp, ns, idxall, g0, g1, o_v, sem0, sem1):
        wid = lax.axis_index("s") * 2 + lax.axis_index("c")
        zero = jnp.zeros((16,), jnp.float32)

        def zbody(r, carry):
            for f in range(Fc // 16):
                o_v[r, pl.ds(16 * f, 16)] = zero
            return carry

        lax.fori_loop(0, 125, zbody, 0)
        pltpu.sync_copy(o_v.at[pl.ds(0, 125)], ns.at[pl.ds(wid * 125, 125)])

        def compute_chunk(d, c, gb, rows_c):
            g_v = g1 if gb else g0
            row0 = DEG_STARTS[d] + wid * ROWS_PER_TILE

            @plsc.parallel_loop(0, rows_c, unroll=2)
            def body(r):
                for f in range(Fc // 16):
                    sl = pl.ds(16 * f, 16)
                    acc = g_v[r * d, sl]
                    for j in range(1, d):
                        acc = acc + g_v[r * d + j, sl]
                    o_v[r, sl] = acc

            pltpu.sync_copy(o_v.at[pl.ds(0, rows_c)],
                            ns.at[pl.ds(row0 + c * R_D[d], rows_c)])

        _pipelined_degs(table, idxp, wid, idxall, (g0, g1), (sem0, sem1),
                        compute_chunk)

    return gather_sum


_gather_sum_128 = _make_gather_sum(128)
_gather_sum_64 = _make_gather_sum(64)


BLK = 4000
NBLK = N // BLK


def _widx(b):
    return jnp.where(b == 0, 0, (b - 1) // 4 + 1)


def _gcmm_body(ns_ref, self_ref, wn_ref, ws_ref, bias_ref, gamma_ref, beta_ref,
               h_ref, so_ref, acc_ref):
    b = pl.program_id(0)
    x = (jnp.dot(ns_ref[...], wn_ref[0], preferred_element_type=jnp.float32)
         + jnp.dot(self_ref[...], ws_ref[0], preferred_element_type=jnp.float32)
         + bias_ref[0])
    h = jnp.tanh(x)
    h_ref[...] = h

    @pl.when(b == 0)
    def _():
        acc_ref[...] = jnp.zeros_like(acc_ref)

    acc_ref[0:1, :] += jnp.sum(h, axis=0, keepdims=True)
    acc_ref[1:2, :] += jnp.sum(h * h, axis=0, keepdims=True)

    @pl.when(b == pl.num_programs(0) - 1)
    def _():
        mean = acc_ref[0:1, :] * (1.0 / N)
        var = acc_ref[1:2, :] * (1.0 / N) - mean * mean
        scale = gamma_ref[...] * jax.lax.rsqrt(var + 1e-3)
        so_ref[0:1, :] = scale
        so_ref[1:2, :] = beta_ref[...] - scale * mean


def _graph_conv_mm(ns, selfrows, Wn, Ws, bsum, gamma, beta, Fc):
    return pl.pallas_call(
        _gcmm_body,
        grid=(NBLK,),
        in_specs=[
            pl.BlockSpec((BLK, Fc), lambda b: (b, 0)),
            pl.BlockSpec((BLK, Fc), lambda b: (b, 0)),
            pl.BlockSpec((1, Fc, 64), lambda b: (_widx(b), 0, 0)),
            pl.BlockSpec((1, Fc, 64), lambda b: (_widx(b), 0, 0)),
            pl.BlockSpec((1, 1, 64), lambda b: (_widx(b), 0, 0)),
            pl.BlockSpec((1, 64), lambda b: (0, 0)),
            pl.BlockSpec((1, 64), lambda b: (0, 0)),
        ],
        out_specs=[
            pl.BlockSpec((BLK, 64), lambda b: (b, 0)),
            pl.BlockSpec((2, 64), lambda b: (0, 0)),
        ],
        out_shape=[
            jax.ShapeDtypeStruct((N, 64), jnp.float32),
            jax.ShapeDtypeStruct((2, 64), jnp.float32),
        ],
        scratch_shapes=[pltpu.VMEM((2, 64), jnp.float32)],
    )(ns, selfrows, Wn, Ws, bsum.reshape(7, 1, 64), gamma.reshape(1, 64),
      beta.reshape(1, 64))


def _make_pool(Fc):
    mesh = plsc.VectorSubcoreMesh(core_axis_name="c", subcore_axis_name="s")

    @functools.partial(
        pl.kernel,
        out_type=jax.ShapeDtypeStruct((N, Fc), jnp.float32),
        mesh=mesh,
        compiler_params=pltpu.CompilerParams(use_tc_tiling_on_sc=False),
        scratch_types=[
            pltpu.VMEM((3008,), jnp.int32),
            pltpu.VMEM((128, Fc), jnp.float32),
            pltpu.VMEM((128, Fc), jnp.float32),
            pltpu.VMEM((128, Fc), jnp.float32),
            pltpu.VMEM((128, Fc), jnp.float32),
            pltpu.VMEM((128, Fc), jnp.float32),
            pltpu.VMEM((2, Fc), jnp.float32),
            pltpu.SemaphoreType.DMA,
            pltpu.SemaphoreType.DMA,
            pltpu.SemaphoreType.DMA,
            pltpu.SemaphoreType.DMA,
        ],
    )
    def pool(h, idxp, so, p, idxall, g0, g1, s0, s1, o_v, so_v,
             sem0, sem1, ss0, ss1):
        wid = lax.axis_index("s") * 2 + lax.axis_index("c")
        pltpu.sync_copy(so, so_v)

        pltpu.sync_copy(h.at[pl.ds(wid * 125, 125)], s0.at[pl.ds(0, 125)])

        def z0(r, carry):
            for f in range(Fc // 16):
                sl = pl.ds(16 * f, 16)
                o_v[r, sl] = s0[r, sl] * so_v[0, sl] + so_v[1, sl]
            return carry

        lax.fori_loop(0, 125, z0, 0)
        pltpu.sync_copy(o_v.at[pl.ds(0, 125)], p.at[pl.ds(wid * 125, 125)])

        def prefetch_self(d, c, sb):
            row0 = DEG_STARTS[d] + wid * ROWS_PER_TILE
            R = R_D[d]
            pltpu.async_copy(h.at[pl.ds(row0 + c * R, R)],
                             (s1 if sb else s0).at[pl.ds(0, R)],
                             ss1 if sb else ss0)

        def compute_chunk(d, c, gb, rows_c):
            g_v = g0 if gb == 0 else g1
            s_v = s0 if gb == 0 else s1
            ssem = ss0 if gb == 0 else ss1
            R = R_D[d]
            row0 = DEG_STARTS[d] + wid * ROWS_PER_TILE
            pltpu.make_async_copy(h.at[pl.ds(row0, R)],
                                  s_v.at[pl.ds(0, R)], ssem).wait()

            @plsc.parallel_loop(0, rows_c, unroll=2)
            def body(r):
                for f in range(Fc // 16):
                    sl = pl.ds(16 * f, 16)
                    m = s_v[r, sl]
                    for j in range(d):
                        m = jnp.maximum(m, g_v[r * d + j, sl])
                    o_v[r, sl] = m * so_v[0, sl] + so_v[1, sl]
            pltpu.sync_copy(o_v.at[pl.ds(0, rows_c)],
                            p.at[pl.ds(row0 + c * R, rows_c)])

        _pipelined_degs(h, idxp, wid, idxall, (g0, g1), (sem0, sem1),
                        compute_chunk, prefetch_self=prefetch_self)

    return pool


_pool_64 = _make_pool(64)


def _split_weights(W, b, Fc):
    wn = jnp.concatenate([jnp.zeros((1, Fc, 64), jnp.float32), W[0:11:2]], axis=0)
    ws = jnp.concatenate([W[12:13], W[1:12:2]], axis=0)
    bs = jnp.concatenate([b[12:13], b[0:11:2] + b[1:12:2]], axis=0)
    return wn, ws, bs


def _d1_body(x_ref, w_ref, b_ref, gamma_ref, beta_ref, h_ref, so_ref, acc_ref):
    b = pl.program_id(0)
    h = jnp.tanh(jnp.dot(x_ref[...], w_ref[...], preferred_element_type=jnp.float32)
                 + b_ref[...])
    h_ref[...] = h

    @pl.when(b == 0)
    def _():
        acc_ref[...] = jnp.zeros_like(acc_ref)

    acc_ref[0:1, :] += jnp.sum(h, axis=0, keepdims=True)
    acc_ref[1:2, :] += jnp.sum(h * h, axis=0, keepdims=True)

    @pl.when(b == pl.num_programs(0) - 1)
    def _():
        mean = acc_ref[0:1, :] * (1.0 / N)
        var = acc_ref[1:2, :] * (1.0 / N) - mean * mean
        scale = gamma_ref[...] * jax.lax.rsqrt(var + 1e-3)
        so_ref[0:1, :] = scale
        so_ref[1:2, :] = beta_ref[...] - scale * mean


def _dense1(x, W, bias, gamma, beta):
    return pl.pallas_call(
        _d1_body,
        grid=(NBLK,),
        in_specs=[
            pl.BlockSpec((BLK, 64), lambda b: (b, 0)),
            pl.BlockSpec((64, 128), lambda b: (0, 0)),
            pl.BlockSpec((1, 128), lambda b: (0, 0)),
            pl.BlockSpec((1, 128), lambda b: (0, 0)),
            pl.BlockSpec((1, 128), lambda b: (0, 0)),
        ],
        out_specs=[
            pl.BlockSpec((BLK, 128), lambda b: (b, 0)),
            pl.BlockSpec((2, 128), lambda b: (0, 0)),
        ],
        out_shape=[
            jax.ShapeDtypeStruct((N, 128), jnp.float32),
            jax.ShapeDtypeStruct((2, 128), jnp.float32),
        ],
        scratch_shapes=[pltpu.VMEM((2, 128), jnp.float32)],
    )(x, W, bias.reshape(1, 128), gamma.reshape(1, 128), beta.reshape(1, 128))


SEGB = 2048
DUMMY = BATCH
_SUM_SPAN = 3200
_FULL_CHUNKS = N // 128
_TAIL = N - _FULL_CHUNKS * 128


def _make_segment():
    mesh = plsc.VectorSubcoreMesh(core_axis_name="c", subcore_axis_name="s")

    @functools.partial(
        pl.kernel,
        out_type=[
            jax.ShapeDtypeStruct((2, SEGB, 128), jnp.float32),
            jax.ShapeDtypeStruct((8, SEGB, 128), jnp.float32),
        ],
        mesh=mesh,
        compiler_params=pltpu.CompilerParams(use_tc_tiling_on_sc=False),
        scratch_types=[
            pltpu.VMEM((128, 128), jnp.float32),
            pltpu.VMEM((128, 128), jnp.float32),
            pltpu.VMEM((128, 32), jnp.float32),
            pltpu.VMEM((128, 32), jnp.float32),
            pltpu.VMEM((128,), jnp.int32),
            pltpu.VMEM((128,), jnp.int32),
            pltpu.VMEM((SEGB, 32), jnp.float32),
            pltpu.VMEM((2, 128), jnp.float32),
            pltpu.VMEM_SHARED((SEGB, 128), jnp.float32),
            pltpu.SemaphoreType.DMA,
            pltpu.SemaphoreType.DMA,
            pltpu.SemaphoreType.DMA,
            pltpu.SemaphoreType.DMA,
        ],
    )
    def segment(h3, mem, so, ssum_out, mx_out,
                x0, x1, xb0, xb1, m0, m1, acc_v, so_v, shared,
                sx0, sx1, sm0, sm1):
        cid = lax.axis_index("c")
        sid = lax.axis_index("s")
        wid = sid * 2 + cid
        pltpu.sync_copy(so, so_v)

        zero = jnp.zeros((16,), jnp.float32)

        def zx(r, carry):
            for f in range(8):
                x0[r, pl.ds(16 * f, 16)] = zero
            return carry

        lax.fori_loop(0, 128, zx, 0)
        pltpu.sync_copy(x0, shared.at[pl.ds(sid * 128, 128)])
        plsc.subcore_barrier()

        xbufs = (x0, x1)
        mbufs = (m0, m1)
        sxs = (sx0, sx1)
        sms = (sm0, sm1)

        def startA(base, b, nrows):
            pltpu.async_copy(mem.at[pl.ds(base, nrows)],
                             mbufs[b].at[pl.ds(0, nrows)], sms[b])
            pltpu.async_copy(h3.at[pl.ds(base, nrows)],
                             xbufs[b].at[pl.ds(0, nrows)], sxs[b])

        def computeA(b, nrows):
            x_v, mem_v = xbufs[b], mbufs[b]
            pltpu.make_async_copy(mem.at[pl.ds(0, nrows)],
                                  mem_v.at[pl.ds(0, nrows)], sms[b]).wait()
            pltpu.make_async_copy(h3.at[pl.ds(0, nrows)],
                                  x_v.at[pl.ds(0, nrows)], sxs[b]).wait()
            if nrows < 128:
                dummy = jnp.full((16,), DUMMY, jnp.int32)
                for k in range((128 - nrows) // 16):
                    mem_v[pl.ds(nrows + 16 * k, 16)] = dummy

            @plsc.parallel_loop(0, 128, unroll=2)
            def abody(r):
                for f in range(8):
                    sl = pl.ds(16 * f, 16)
                    x_v[r, sl] = x_v[r, sl] * so_v[0, sl] + so_v[1, sl]
            pltpu.sync_copy(x_v, shared.at[mem_v], add=True)

        def pipeline2(nchunk, last_rows, start_fn, compute_fn):
            size = lambda c: last_rows if c == nchunk - 1 else 128
            if nchunk == 1:
                start_fn(0, 0, last_rows)
                compute_fn(0, last_rows)
                return
            pairs = ((nchunk - 1) // 2 if last_rows == 128
                     else max(0, (nchunk - 2) // 2))
            start_fn(0, 0, 128)
            if pairs:
                def pb(i, carry):
                    c0 = 2 * i
                    start_fn(c0 + 1, 1, 128)
                    compute_fn(0, 128)
                    start_fn(c0 + 2, 0, 128)
                    compute_fn(1, 128)
                    return carry

                lax.fori_loop(0, pairs, pb, 0)
            t0 = 2 * pairs
            remc = nchunk - t0
            if remc == 1:
                compute_fn(0, size(t0))
            elif remc == 2:
                start_fn(t0 + 1, 1, size(t0 + 1))
                compute_fn(0, 128)
                compute_fn(1, size(t0 + 1))
            else:
                start_fn(t0 + 1, 1, 128)
                compute_fn(0, 128)
                start_fn(t0 + 2, 0, size(t0 + 2))
                compute_fn(1, 128)
                compute_fn(0, size(t0 + 2))

        def phaseA(nchunk, base0, last_rows):
            pipeline2(nchunk, last_rows,
                      lambda c, b, nr: startA(base0 + c * 128, b, nr),
                      computeA)

        @pl.when(wid < 31)
        def _():
            phaseA(25, wid * _SUM_SPAN, 128)

        @pl.when(wid == 31)
        def _():
            phaseA(7, 31 * _SUM_SPAN, _TAIL)

        plsc.subcore_barrier()
        pltpu.sync_copy(shared.at[pl.ds(sid * 128, 128)],
                        ssum_out.at[cid, pl.ds(sid * 128, 128)])

        rg = wid // 4
        fg = wid % 4
        fof = fg * 32
        ninf = jnp.full((16,), -jnp.inf, jnp.float32)

        def zacc(row, carry):
            acc_v[row, pl.ds(0, 16)] = ninf
            acc_v[row, pl.ds(16, 16)] = ninf
            return carry

        lax.fori_loop(0, SEGB, zacc, 0)

        xbbufs = (xb0, xb1)

        def startB(base, b, nrows):
            pltpu.async_copy(mem.at[pl.ds(base, nrows)],
                             mbufs[b].at[pl.ds(0, nrows)], sms[b])
            pltpu.async_copy(h3.at[pl.ds(base, nrows), pl.ds(fof, 32)],
                             xbbufs[b].at[pl.ds(0, nrows)], sxs[b])

        def computeB(b, nrows):
            xb_v, mem_v = xbbufs[b], mbufs[b]
            pltpu.make_async_copy(mem.at[pl.ds(0, nrows)],
                                  mem_v.at[pl.ds(0, nrows)], sms[b]).wait()
            pltpu.make_async_copy(h3.at[pl.ds(0, nrows), pl.ds(0, 32)],
                                  xb_v.at[pl.ds(0, nrows)], sxs[b]).wait()

            def mbody(g, carry):
                mv = mem_v[pl.ds(g * 16, 16)]
                for k in range(16):
                    m = mv[k]
                    r = g * 16 + k
                    for f2 in range(2):
                        sl = pl.ds(16 * f2, 16)
                        acc_v[m, sl] = jnp.maximum(acc_v[m, sl], xb_v[r, sl])
                return carry

            lax.fori_loop(0, nrows // 16, mbody, 0)

        def phaseB(nchunk, coff, last_rows):
            pipeline2(nchunk, last_rows,
                      lambda c, b, nr: startB((coff + c) * 128, b, nr),
                      computeB)

        @pl.when(rg < 7)
        def _():
            phaseB(98, rg * 98, 128)

        @pl.when(rg == 7)
        def _():
            phaseB(96, 686, _TAIL)

        def affb(row, carry):
            for f2 in range(2):
                sl = pl.ds(16 * f2, 16)
                fsl = pl.ds(fof + 16 * f2, 16)
                acc_v[row, sl] = acc_v[row, sl] * so_v[0, fsl] + so_v[1, fsl]
            return carry

        lax.fori_loop(0, SEGB, affb, 0)
        pltpu.sync_copy(acc_v, mx_out.at[rg, pl.ds(0, SEGB), pl.ds(fof, 32)])

    return segment


_segment = _make_segment()


def _final_body(ssum_ref, mx_ref, d2w_ref, d2b_ref, d3w_ref, d3b_ref, out_ref):
    s = ssum_ref[0] + ssum_ref[1]
    m = jnp.max(mx_ref[...], axis=0)
    g = jnp.tanh(jnp.concatenate([s[0:BATCH], m[0:BATCH]], axis=1))
    z = jax.nn.sigmoid(jnp.dot(g, d2w_ref[...], preferred_element_type=jnp.float32)
                       + d2b_ref[...])
    out_ref[...] = (jnp.dot(z, d3w_ref[...], preferred_element_type=jnp.float32)
                    + d3b_ref[...])


def kernel(atom_features, degree_slice, membership, deg_adj_1, deg_adj_2, deg_adj_3, deg_adj_4, deg_adj_5, deg_adj_6, gc1_W, gc1_b, gc2_W, gc2_b, bn1_gamma, bn1_beta, bn3_gamma, bn3_beta, d1_W, d1_b, d2_W, d2_b, d3_W, d3_b):
    deg_adj_lists = [deg_adj_1, deg_adj_2, deg_adj_3, deg_adj_4, deg_adj_5, deg_adj_6]
    idxp = _build_padded_idx(deg_adj_lists)

    wn1, ws1, bs1 = _split_weights(gc1_W, gc1_b, F)
    wn2, ws2, bs2 = _split_weights(gc2_W, gc2_b, 64)

    ns1 = _gather_sum_128(atom_features, idxp)
    h1, so1 = _graph_conv_mm(ns1, atom_features, wn1, ws1, bs1, bn1_gamma, bn1_beta, F)
    p1 = _pool_64(h1, idxp, so1)
    ns2 = _gather_sum_64(p1, idxp)
    h2, so2 = _graph_conv_mm(ns2, p1, wn2, ws2, bs2, bn1_gamma, bn1_beta, 64)
    p2 = _pool_64(h2, idxp, so2)
    h3, so3 = _dense1(p2, d1_W, d1_b, bn3_gamma, bn3_beta)
    ssum, mx = _segment(h3, membership, so3)
    out = pl.pallas_call(
        _final_body,
        out_shape=jax.ShapeDtypeStruct((BATCH, 1), jnp.float32),
    )(ssum, mx, d2_W, d2_b.reshape(1, 64), d3_W, d3_b.reshape(1, 1))
    return out
